# baseline (speedup 1.0000x reference)
import functools

import jax
import jax.numpy as jnp
from jax import lax
from jax.experimental import pallas as pl
from jax.experimental.pallas import tpu as pltpu
from jax.experimental.pallas import tpu_sc as plsc

B = 16384
EMB = 32
EPS = 1e-3
NC = 2
NS = 16
FPS = EMB // NC
FPP = FPS // 2
BPT = B // NS
UROW = 100096
HROW = UROW // 2
UCH = 4352
OROW = 1008
ICH = 128
NCH = BPT // ICH

BLKC = 2048


def _sc_gather_body(ut_hbm, ot_hbm, cid_hbm, oid_hbm, ce_hbm, oe_hbm,
                    idx_u, idx_o, dst_u, dst_o, ushr, oshr, sem, gsem):
    s = lax.axis_index("c")
    t = lax.axis_index("s")
    col0 = t * BPT

    pltpu.sync_copy(cid_hbm.at[pl.ds(col0, BPT)], idx_u)
    pltpu.sync_copy(oid_hbm.at[pl.ds(col0, BPT)], idx_o)
    pltpu.async_copy(ot_hbm.at[pl.ds((s * FPS + t) * OROW, OROW)],
                     oshr.at[t], sem).wait()

    for p in range(2):
        fl_stage = t // 2
        half = t % 2
        pltpu.async_copy(
            ut_hbm.at[pl.ds((s * FPS + p * FPP + fl_stage) * UROW
                            + half * HROW, HROW)],
            ushr.at[fl_stage, pl.ds(half * HROW, HROW)], sem).wait()
        plsc.subcore_barrier()

        for fl in range(FPP):
            cps = []
            for k in range(NCH):
                cps.append(pltpu.async_copy(
                    ushr.at[fl].at[idx_u.at[pl.ds(k * ICH, ICH)]],
                    dst_u.at[fl, pl.ds(k * ICH, ICH)], gsem))
                cps.append(pltpu.async_copy(
                    oshr.at[p * FPP + fl].at[idx_o.at[pl.ds(k * ICH, ICH)]],
                    dst_o.at[fl, pl.ds(k * ICH, ICH)], gsem))
            for cp in cps:
                cp.wait()

        row0 = s * FPS + p * FPP
        for fl in range(FPP):
            pltpu.sync_copy(dst_u.at[fl],
                            ce_hbm.at[pl.ds((row0 + fl) * B + col0, BPT)])
            pltpu.sync_copy(dst_o.at[fl],
                            oe_hbm.at[pl.ds((row0 + fl) * B + col0, BPT)])
        plsc.subcore_barrier()


def _sc_gather(ut, ot, customer_id, operator_name):
    mesh = plsc.VectorSubcoreMesh(core_axis_name="c", subcore_axis_name="s",
                                  num_cores=NC, num_subcores=NS)
    return pl.kernel(
        _sc_gather_body,
        out_type=(jax.ShapeDtypeStruct((EMB * B,), jnp.float32),
                  jax.ShapeDtypeStruct((EMB * B,), jnp.float32)),
        mesh=mesh,
        scratch_types=[
            pltpu.VMEM((BPT,), jnp.int32),
            pltpu.VMEM((BPT,), jnp.int32),
            pltpu.VMEM((FPP, BPT), jnp.float32),
            pltpu.VMEM((FPP, BPT), jnp.float32),
            pltpu.VMEM_SHARED((FPP, UROW), jnp.float32),
            pltpu.VMEM_SHARED((FPS, OROW), jnp.float32),
            pltpu.SemaphoreType.DMA,
            pltpu.SemaphoreType.DMA,
        ],
        compiler_params=pltpu.CompilerParams(use_tc_tiling_on_sc=False),
    )(ut, ot, customer_id, operator_name)


def _repack_body(in_ref, out_ref):
    out_ref[...] = in_ref[...]


def _repack(ut):
    return pl.pallas_call(
        _repack_body,
        grid=(UROW // UCH,),
        in_specs=[pl.BlockSpec((EMB, UCH), lambda i: (0, i))],
        out_specs=pl.BlockSpec((EMB, UCH), lambda i: (0, i)),
        out_shape=jax.ShapeDtypeStruct((EMB, UROW), jnp.float32),
    )(ut).reshape(-1)


def _mlp_body(xc_ref, xo_ref, W1_ref, b1_ref, g1_ref, be1_ref, m1_ref, v1_ref,
              W2_ref, b2_ref, g2_ref, be2_ref, m2_ref, v2_ref,
              W3_ref, b3_ref, out_ref):
    s1 = g1_ref[...] * lax.rsqrt(v1_ref[...] + EPS)
    c1 = (b1_ref[...] - m1_ref[...]) * s1 + be1_ref[...]
    s2 = g2_ref[...] * lax.rsqrt(v2_ref[...] + EPS)
    c2 = (b2_ref[...] - m2_ref[...]) * s2 + be2_ref[...]

    W1 = W1_ref[...]
    cn = (((0,), (0,)), ((), ()))
    acc = lax.dot_general(W1[:EMB, :], xc_ref[...], cn,
                          preferred_element_type=jnp.float32)
    acc += lax.dot_general(W1[EMB:, :], xo_ref[...], cn,
                           preferred_element_type=jnp.float32)
    h1 = jnp.maximum(acc * s1 + c1, 0.0)
    h2 = jnp.maximum(
        lax.dot_general(W2_ref[...], h1, cn,
                        preferred_element_type=jnp.float32) * s2 + c2, 0.0)
    out_ref[...] = (lax.dot_general(W3_ref[...], h2, cn,
                                    preferred_element_type=jnp.float32)
                    + b3_ref[...])


def _mlp(xc, xo, W1, b1, g1, be1, m1, v1, W2, b2, g2, be2, m2, v2, W3, b3):
    grid = (B // BLKC,)
    full = lambda shape: pl.BlockSpec(shape, lambda i: (0, 0))
    return pl.pallas_call(
        _mlp_body,
        grid=grid,
        in_specs=[
            pl.BlockSpec((EMB, BLKC), lambda i: (0, i)),
            pl.BlockSpec((EMB, BLKC), lambda i: (0, i)),
            full((2 * EMB, 256)), full((256, 1)), full((256, 1)),
            full((256, 1)), full((256, 1)), full((256, 1)),
            full((256, 128)), full((128, 1)), full((128, 1)),
            full((128, 1)), full((128, 1)), full((128, 1)),
            full((128, 1)), full((1, 1)),
        ],
        out_specs=pl.BlockSpec((1, BLKC), lambda i: (0, i)),
        out_shape=jax.ShapeDtypeStruct((1, B), jnp.float32),
    )(xc, xo, W1, b1.reshape(-1, 1), g1.reshape(-1, 1), be1.reshape(-1, 1),
      m1.reshape(-1, 1), v1.reshape(-1, 1), W2, b2.reshape(-1, 1),
      g2.reshape(-1, 1), be2.reshape(-1, 1), m2.reshape(-1, 1),
      v2.reshape(-1, 1), W3, b3.reshape(1, 1))


def kernel(customer_id, operator_name, user_table, op_table,
           W1, b1, g1, be1, m1, v1, W2, b2, g2, be2, m2, v2, W3, b3):
    ut = _repack(user_table.T)
    ot = jnp.pad(op_table.T,
                 ((0, 0), (0, OROW - op_table.shape[0]))).reshape(-1)
    ce_f, oe_f = _sc_gather(ut, ot,
                            customer_id.astype(jnp.int32),
                            operator_name.astype(jnp.int32))
    out = _mlp(ce_f.reshape(EMB, B), oe_f.reshape(EMB, B),
               W1, b1, g1, be1, m1, v1,
               W2, b2, g2, be2, m2, v2, W3, b3)
    return out.reshape(B, 1)


# speedup vs baseline: 1.0979x; 1.0979x over previous
import functools

import jax
import jax.numpy as jnp
from jax import lax
from jax.experimental import pallas as pl
from jax.experimental.pallas import tpu as pltpu
from jax.experimental.pallas import tpu_sc as plsc

B = 16384
EMB = 32
EPS = 1e-3
NC = 2
NS = 16
FPS = EMB // NC
FPP = FPS // 2
BPT = B // NS
UROW = 100096
HROW = UROW // 2
UCH = 4352
OROW = 1008
ICH = 128
NCH = BPT // ICH

BLKC = 2048


def _sc_gather_body(ut_hbm, ot_hbm, cid_hbm, oid_hbm, ce_hbm, oe_hbm,
                    idx_u, idx_o, dst_u, dst_o, ushr, oshr, sem, gsem):
    s = lax.axis_index("c")
    t = lax.axis_index("s")
    col0 = t * BPT

    pltpu.sync_copy(cid_hbm.at[pl.ds(col0, BPT)], idx_u)
    pltpu.sync_copy(oid_hbm.at[pl.ds(col0, BPT)], idx_o)
    pltpu.async_copy(ot_hbm.at[pl.ds((s * FPS + t) * OROW, OROW)],
                     oshr.at[t], sem).wait()

    for p in range(2):
        fl_stage = t // 2
        half = t % 2
        pltpu.async_copy(
            ut_hbm.at[pl.ds((s * FPS + p * FPP + fl_stage) * UROW
                            + half * HROW, HROW)],
            ushr.at[fl_stage, pl.ds(half * HROW, HROW)], sem).wait()
        plsc.subcore_barrier()

        for fl in range(FPP):
            cps = []
            for k in range(NCH):
                cps.append(pltpu.async_copy(
                    ushr.at[fl].at[idx_u.at[pl.ds(k * ICH, ICH)]],
                    dst_u.at[fl, pl.ds(k * ICH, ICH)], gsem))
                cps.append(pltpu.async_copy(
                    oshr.at[p * FPP + fl].at[idx_o.at[pl.ds(k * ICH, ICH)]],
                    dst_o.at[fl, pl.ds(k * ICH, ICH)], gsem))
            for cp in cps:
                cp.wait()

        row0 = s * FPS + p * FPP
        for fl in range(FPP):
            pltpu.sync_copy(dst_u.at[fl],
                            ce_hbm.at[pl.ds((row0 + fl) * B + col0, BPT)])
            pltpu.sync_copy(dst_o.at[fl],
                            oe_hbm.at[pl.ds((row0 + fl) * B + col0, BPT)])
        plsc.subcore_barrier()


def _sc_gather(ut, ot, customer_id, operator_name):
    mesh = plsc.VectorSubcoreMesh(core_axis_name="c", subcore_axis_name="s",
                                  num_cores=NC, num_subcores=NS)
    return pl.kernel(
        _sc_gather_body,
        out_type=(jax.ShapeDtypeStruct((EMB * B,), jnp.float32),
                  jax.ShapeDtypeStruct((EMB * B,), jnp.float32)),
        mesh=mesh,
        scratch_types=[
            pltpu.VMEM((BPT,), jnp.int32),
            pltpu.VMEM((BPT,), jnp.int32),
            pltpu.VMEM((FPP, BPT), jnp.float32),
            pltpu.VMEM((FPP, BPT), jnp.float32),
            pltpu.VMEM_SHARED((FPP, UROW), jnp.float32),
            pltpu.VMEM_SHARED((FPS, OROW), jnp.float32),
            pltpu.SemaphoreType.DMA,
            pltpu.SemaphoreType.DMA,
        ],
        compiler_params=pltpu.CompilerParams(use_tc_tiling_on_sc=False),
    )(ut, ot, customer_id, operator_name)


def _repack_body(in_ref, out_ref):
    out_ref[...] = in_ref[...]


def _repack(ut):
    return pl.pallas_call(
        _repack_body,
        grid=(UROW // UCH,),
        in_specs=[pl.BlockSpec((EMB, UCH), lambda i: (0, i))],
        out_specs=pl.BlockSpec((EMB, UCH), lambda i: (0, i)),
        out_shape=jax.ShapeDtypeStruct((EMB, UROW), jnp.float32),
    )(ut).reshape(-1)


def _mlp_body(xc_ref, xo_ref, W1_ref, b1_ref, g1_ref, be1_ref, m1_ref, v1_ref,
              W2_ref, b2_ref, g2_ref, be2_ref, m2_ref, v2_ref,
              W3_ref, b3_ref, out_ref):
    s1 = g1_ref[...] * lax.rsqrt(v1_ref[...] + EPS)
    c1 = (b1_ref[...] - m1_ref[...]) * s1 + be1_ref[...]
    s2 = g2_ref[...] * lax.rsqrt(v2_ref[...] + EPS)
    c2 = (b2_ref[...] - m2_ref[...]) * s2 + be2_ref[...]

    W1 = W1_ref[...]
    cn = (((0,), (0,)), ((), ()))
    acc = lax.dot_general(W1[:EMB, :], xc_ref[...], cn,
                          preferred_element_type=jnp.float32)
    acc += lax.dot_general(W1[EMB:, :], xo_ref[...], cn,
                           preferred_element_type=jnp.float32)
    h1 = jnp.maximum(acc * s1 + c1, 0.0)
    h2 = jnp.maximum(
        lax.dot_general(W2_ref[...], h1, cn,
                        preferred_element_type=jnp.float32) * s2 + c2, 0.0)
    out_ref[...] = (lax.dot_general(W3_ref[...], h2, cn,
                                    preferred_element_type=jnp.float32)
                    + b3_ref[...])


def _mlp(xc, xo, W1, b1, g1, be1, m1, v1, W2, b2, g2, be2, m2, v2, W3, b3):
    grid = (B // BLKC,)
    full = lambda shape: pl.BlockSpec(shape, lambda i: (0, 0))
    return pl.pallas_call(
        _mlp_body,
        grid=grid,
        in_specs=[
            pl.BlockSpec((EMB, BLKC), lambda i: (0, i)),
            pl.BlockSpec((EMB, BLKC), lambda i: (0, i)),
            full((2 * EMB, 256)), full((256, 1)), full((256, 1)),
            full((256, 1)), full((256, 1)), full((256, 1)),
            full((256, 128)), full((128, 1)), full((128, 1)),
            full((128, 1)), full((128, 1)), full((128, 1)),
            full((128, 1)), full((1, 1)),
        ],
        out_specs=pl.BlockSpec((1, BLKC), lambda i: (0, i)),
        out_shape=jax.ShapeDtypeStruct((1, B), jnp.float32),
    )(xc, xo, W1, b1.reshape(-1, 1), g1.reshape(-1, 1), be1.reshape(-1, 1),
      m1.reshape(-1, 1), v1.reshape(-1, 1), W2, b2.reshape(-1, 1),
      g2.reshape(-1, 1), be2.reshape(-1, 1), m2.reshape(-1, 1),
      v2.reshape(-1, 1), W3, b3.reshape(1, 1))


def kernel(customer_id, operator_name, user_table, op_table,
           W1, b1, g1, be1, m1, v1, W2, b2, g2, be2, m2, v2, W3, b3):
    ut = jnp.pad(user_table.T,
                 ((0, 0), (0, UROW - user_table.shape[0]))).reshape(-1)
    ot = jnp.pad(op_table.T,
                 ((0, 0), (0, OROW - op_table.shape[0]))).reshape(-1)
    ce_f, oe_f = _sc_gather(ut, ot,
                            customer_id.astype(jnp.int32),
                            operator_name.astype(jnp.int32))
    out = _mlp(ce_f.reshape(EMB, B), oe_f.reshape(EMB, B),
               W1, b1, g1, be1, m1, v1,
               W2, b2, g2, be2, m2, v2, W3, b3)
    return out.reshape(B, 1)


# speedup vs baseline: 1.0989x; 1.0010x over previous
import functools

import jax
import jax.numpy as jnp
from jax import lax
from jax.experimental import pallas as pl
from jax.experimental.pallas import tpu as pltpu
from jax.experimental.pallas import tpu_sc as plsc

B = 16384
EMB = 32
EPS = 1e-3
NC = 2
NS = 16
FPS = EMB // NC
FPP = FPS // 2
BPT = B // NS
UROW = 100096
HROW = UROW // 2
OROW = 1008
ICH = 128
NCH = BPT // ICH

BLKC = 2048


def _sc_gather_body(ut_hbm, ot_hbm, cid_hbm, oid_hbm, ce_hbm, oe_hbm,
                    idx_u, idx_o, dst_u, dst_o, ushr, oshr, sem, gsem):
    s = lax.axis_index("c")
    t = lax.axis_index("s")
    col0 = t * BPT

    pltpu.sync_copy(cid_hbm.at[pl.ds(col0, BPT)], idx_u)
    pltpu.sync_copy(oid_hbm.at[pl.ds(col0, BPT)], idx_o)
    pltpu.async_copy(ot_hbm.at[pl.ds((s * FPS + t) * OROW, OROW)],
                     oshr.at[t], sem).wait()

    for p in range(2):
        fl_stage = t // 2
        half = t % 2
        pltpu.async_copy(
            ut_hbm.at[pl.ds((s * FPS + p * FPP + fl_stage) * UROW
                            + half * HROW, HROW)],
            ushr.at[fl_stage, pl.ds(half * HROW, HROW)], sem).wait()
        plsc.subcore_barrier()

        for fl in range(FPP):
            cps = []
            for k in range(NCH):
                cps.append(pltpu.async_copy(
                    ushr.at[fl].at[idx_u.at[pl.ds(k * ICH, ICH)]],
                    dst_u.at[fl, pl.ds(k * ICH, ICH)], gsem))
                cps.append(pltpu.async_copy(
                    oshr.at[p * FPP + fl].at[idx_o.at[pl.ds(k * ICH, ICH)]],
                    dst_o.at[fl, pl.ds(k * ICH, ICH)], gsem))
            for cp in cps:
                cp.wait()

        row0 = s * FPS + p * FPP
        for fl in range(FPP):
            pltpu.sync_copy(dst_u.at[fl],
                            ce_hbm.at[pl.ds((row0 + fl) * B + col0, BPT)])
            pltpu.sync_copy(dst_o.at[fl],
                            oe_hbm.at[pl.ds((row0 + fl) * B + col0, BPT)])
        plsc.subcore_barrier()


def _sc_gather(ut, ot, customer_id, operator_name):
    mesh = plsc.VectorSubcoreMesh(core_axis_name="c", subcore_axis_name="s",
                                  num_cores=NC, num_subcores=NS)
    return pl.kernel(
        _sc_gather_body,
        out_type=(jax.ShapeDtypeStruct((EMB * B,), jnp.float32),
                  jax.ShapeDtypeStruct((EMB * B,), jnp.float32)),
        mesh=mesh,
        scratch_types=[
            pltpu.VMEM((BPT,), jnp.int32),
            pltpu.VMEM((BPT,), jnp.int32),
            pltpu.VMEM((FPP, BPT), jnp.float32),
            pltpu.VMEM((FPP, BPT), jnp.float32),
            pltpu.VMEM_SHARED((FPP, UROW), jnp.float32),
            pltpu.VMEM_SHARED((FPS, OROW), jnp.float32),
            pltpu.SemaphoreType.DMA,
            pltpu.SemaphoreType.DMA,
        ],
        compiler_params=pltpu.CompilerParams(use_tc_tiling_on_sc=False),
    )(ut, ot, customer_id, operator_name)


def _mlp_body(xc_ref, xo_ref, W1_ref, b1_ref, g1_ref, be1_ref, m1_ref, v1_ref,
              W2_ref, b2_ref, g2_ref, be2_ref, m2_ref, v2_ref,
              W3_ref, b3_ref, out_ref):
    s1 = g1_ref[...] * lax.rsqrt(v1_ref[...] + EPS)
    c1 = (b1_ref[...] - m1_ref[...]) * s1 + be1_ref[...]
    s2 = g2_ref[...] * lax.rsqrt(v2_ref[...] + EPS)
    c2 = (b2_ref[...] - m2_ref[...]) * s2 + be2_ref[...]

    W1 = W1_ref[...]
    cn = (((0,), (0,)), ((), ()))
    acc = lax.dot_general(W1[:EMB, :], xc_ref[...], cn,
                          preferred_element_type=jnp.float32)
    acc += lax.dot_general(W1[EMB:, :], xo_ref[...], cn,
                           preferred_element_type=jnp.float32)
    h1 = jnp.maximum(acc * s1 + c1, 0.0)
    h2 = jnp.maximum(
        lax.dot_general(W2_ref[...], h1, cn,
                        preferred_element_type=jnp.float32) * s2 + c2, 0.0)
    out_ref[...] = (lax.dot_general(W3_ref[...], h2, cn,
                                    preferred_element_type=jnp.float32)
                    + b3_ref[...])


def _mlp(xc, xo, W1, b1, g1, be1, m1, v1, W2, b2, g2, be2, m2, v2, W3, b3):
    grid = (B // BLKC,)
    full = lambda shape: pl.BlockSpec(shape, lambda i: (0, 0))
    return pl.pallas_call(
        _mlp_body,
        grid=grid,
        in_specs=[
            pl.BlockSpec((EMB, BLKC), lambda i: (0, i)),
            pl.BlockSpec((EMB, BLKC), lambda i: (0, i)),
            full((2 * EMB, 256)), full((256, 1)), full((256, 1)),
            full((256, 1)), full((256, 1)), full((256, 1)),
            full((256, 128)), full((128, 1)), full((128, 1)),
            full((128, 1)), full((128, 1)), full((128, 1)),
            full((128, 1)), full((1, 1)),
        ],
        out_specs=pl.BlockSpec((1, BLKC), lambda i: (0, i)),
        out_shape=jax.ShapeDtypeStruct((1, B), jnp.float32),
    )(xc, xo, W1, b1.reshape(-1, 1), g1.reshape(-1, 1), be1.reshape(-1, 1),
      m1.reshape(-1, 1), v1.reshape(-1, 1), W2, b2.reshape(-1, 1),
      g2.reshape(-1, 1), be2.reshape(-1, 1), m2.reshape(-1, 1),
      v2.reshape(-1, 1), W3, b3.reshape(1, 1))


def kernel(customer_id, operator_name, user_table, op_table,
           W1, b1, g1, be1, m1, v1, W2, b2, g2, be2, m2, v2, W3, b3):
    ut = jnp.pad(user_table.T,
                 ((0, 0), (0, UROW - user_table.shape[0]))).reshape(-1)
    ot = jnp.pad(op_table.T,
                 ((0, 0), (0, OROW - op_table.shape[0]))).reshape(-1)
    ce_f, oe_f = _sc_gather(ut, ot,
                            customer_id.astype(jnp.int32),
                            operator_name.astype(jnp.int32))
    out = _mlp(ce_f.reshape(EMB, B), oe_f.reshape(EMB, B),
               W1, b1, g1, be1, m1, v1,
               W2, b2, g2, be2, m2, v2, W3, b3)
    return out.reshape(B, 1)
